# Initial kernel scaffold; baseline (speedup 1.0000x reference)
import jax
import jax.numpy as jnp
from jax.experimental import pallas as pl


def kernel(x, edge_index, batch, params):
    raise NotImplementedError("write your pallas kernel here")


# speedup vs baseline: 3.5504x; 3.5504x over previous
import functools

import jax
import jax.numpy as jnp
from jax import lax
from jax.experimental import pallas as pl
from jax.experimental.pallas import tpu as pltpu
from jax.experimental.pallas import tpu_sc as plsc

NC = 2
NS = 16
CHUNK = 128
FH = 128


@functools.cache
def _make_edge_agg(n, e_pad):
    epw = e_pad // NS
    nchunks = epw // CHUNK
    rpt8 = (n // NS) // 8 * 8
    last_len = n - (NS - 1) * rpt8
    n_spm = n + 8
    mesh = plsc.VectorSubcoreMesh(
        core_axis_name="c", subcore_axis_name="s",
        num_cores=NC, num_subcores=NS)

    @functools.partial(
        pl.kernel,
        out_type=(jax.ShapeDtypeStruct((n, FH), jnp.float32),
                  jax.ShapeDtypeStruct((n, FH), jnp.float32)),
        mesh=mesh,
        scratch_types=[
            pltpu.VMEM((CHUNK,), jnp.int32),
            pltpu.VMEM((CHUNK,), jnp.int32),
            pltpu.VMEM((CHUNK, FH), jnp.float32),
            pltpu.VMEM_SHARED((n_spm, FH), jnp.float32),
            pltpu.SemaphoreType.DMA,
        ],
    )
    def k(h0_hbm, h1_hbm, src_hbm, dst_hbm, z0_hbm, z1_hbm,
          sidx, didx, gbuf, zsh, sem):
        c = lax.axis_index("c")
        s = lax.axis_index("s")
        row0 = s * rpt8

        def rows_copy(mk):
            @pl.when(s < NS - 1)
            def _():
                mk(row0, rpt8)

            @pl.when(s == NS - 1)
            def _():
                mk((NS - 1) * rpt8, last_len)

        for ci, h_hbm in enumerate((h0_hbm, h1_hbm)):
            @pl.when(c == ci)
            def _(h_hbm=h_hbm):
                rows_copy(lambda r0, ln: pltpu.sync_copy(
                    h_hbm.at[pl.ds(r0, ln)], zsh.at[pl.ds(r0, ln)]))
        plsc.subcore_barrier()
        ebase = s * epw

        def chunk(j, carry):
            off = ebase + j * CHUNK
            pltpu.sync_copy(src_hbm.at[pl.ds(off, CHUNK)], sidx)
            pltpu.sync_copy(dst_hbm.at[pl.ds(off, CHUNK)], didx)
            for ci, h_hbm in enumerate((h0_hbm, h1_hbm)):
                @pl.when(c == ci)
                def _(h_hbm=h_hbm):
                    pltpu.async_copy(h_hbm.at[sidx], gbuf, sem).wait()
            pltpu.sync_copy(gbuf, zsh.at[didx], add=True)
            return carry

        lax.fori_loop(0, nchunks, chunk, 0)
        plsc.subcore_barrier()
        for ci, z_hbm in enumerate((z0_hbm, z1_hbm)):
            @pl.when(c == ci)
            def _(z_hbm=z_hbm):
                rows_copy(lambda r0, ln: pltpu.sync_copy(
                    zsh.at[pl.ds(r0, ln)], z_hbm.at[pl.ds(r0, ln)]))

    return k


@functools.cache
def _make_mlp(n, bm):
    def body(z0, z1, w1, b1, w2, b2, o0, o1):
        y = jnp.dot(z0[...], w1[:FH, :], preferred_element_type=jnp.float32)
        y += jnp.dot(z1[...], w1[FH:, :], preferred_element_type=jnp.float32)
        y = jnp.maximum(y + b1[...], 0.0)
        h = jnp.dot(y, w2[...], preferred_element_type=jnp.float32) + b2[...]
        h = jnp.maximum(h, 0.0)
        o0[...] = h[:, :FH]
        o1[...] = h[:, FH:]

    return pl.pallas_call(
        body,
        grid=(n // bm,),
        in_specs=[
            pl.BlockSpec((bm, FH), lambda i: (i, 0)),
            pl.BlockSpec((bm, FH), lambda i: (i, 0)),
            pl.BlockSpec((2 * FH, 2 * FH), lambda i: (0, 0)),
            pl.BlockSpec((1, 2 * FH), lambda i: (0, 0)),
            pl.BlockSpec((2 * FH, 2 * FH), lambda i: (0, 0)),
            pl.BlockSpec((1, 2 * FH), lambda i: (0, 0)),
        ],
        out_specs=[
            pl.BlockSpec((bm, FH), lambda i: (i, 0)),
            pl.BlockSpec((bm, FH), lambda i: (i, 0)),
        ],
        out_shape=(jax.ShapeDtypeStruct((n, FH), jnp.float32),
                   jax.ShapeDtypeStruct((n, FH), jnp.float32)),
    )


@functools.cache
def _make_pool_head(n, bm, g, d):
    nb = n // bm
    nh = 12

    def body(batch_ref, *refs):
        h_refs = refs[:nh]
        w1, b1, w2, b2, oh, ols, acc = refs[nh:]
        k = pl.program_id(0)

        @pl.when(k == 0)
        def _():
            acc[...] = jnp.zeros_like(acc)

        b = batch_ref[0, 0, :]
        onehot = (b[None, :] ==
                  lax.broadcasted_iota(jnp.int32, (g, bm), 0)
                  ).astype(jnp.float32)
        for idx, href in enumerate(h_refs):
            li, half = divmod(idx, 2)
            col = li * 2 * FH + half * FH
            acc[:, col:col + FH] += jnp.dot(
                onehot, href[...], preferred_element_type=jnp.float32)

        @pl.when(k == nb - 1)
        def _():
            p = acc[...]
            a1 = jnp.dot(p, w1[...], preferred_element_type=jnp.float32)
            a1 = jnp.maximum(a1 + b1[...], 0.0)
            z2 = jnp.dot(a1, w2[...], preferred_element_type=jnp.float32)
            z2 = z2 + b2[...]
            oh[...] = z2
            m = jnp.max(z2, axis=1, keepdims=True)
            ls = z2 - m
            ols[...] = ls - jnp.log(jnp.sum(jnp.exp(ls), axis=1,
                                            keepdims=True))

    h_spec = pl.BlockSpec((bm, FH), lambda i: (i, 0))
    return pl.pallas_call(
        body,
        grid=(nb,),
        in_specs=[pl.BlockSpec((1, 1, bm), lambda i: (i, 0, 0))]
        + [h_spec] * nh
        + [
            pl.BlockSpec((d, d), lambda i: (0, 0)),
            pl.BlockSpec((1, d), lambda i: (0, 0)),
            pl.BlockSpec((d, d), lambda i: (0, 0)),
            pl.BlockSpec((1, d), lambda i: (0, 0)),
        ],
        out_specs=[
            pl.BlockSpec((g, d), lambda i: (0, 0)),
            pl.BlockSpec((g, d), lambda i: (0, 0)),
        ],
        out_shape=(jax.ShapeDtypeStruct((g, d), jnp.float32),
                   jax.ShapeDtypeStruct((g, d), jnp.float32)),
        scratch_shapes=[pltpu.VMEM((g, d), jnp.float32)],
    )


def _fold_bn(p, f_in):
    scale = p["g"] / jnp.sqrt(1.0 + 1e-5)
    w1 = p["W1"] * scale[None, :]
    b1 = p["b1"] * scale + p["bb"]
    if f_in < 2 * FH:
        w1 = jnp.concatenate(
            [w1, jnp.zeros((2 * FH - f_in, w1.shape[1]), w1.dtype)], axis=0)
    return w1, b1.reshape(1, -1)


def kernel(x, edge_index, batch, params):
    n, f_in = x.shape
    e = edge_index.shape[1]
    g = 128
    bm = 1000

    align = NS * CHUNK
    e_pad = ((e + align - 1) // align) * align
    pad = e_pad - e
    src = jnp.concatenate([edge_index[0], jnp.zeros((pad,), jnp.int32)])
    dst = jnp.concatenate([edge_index[1], jnp.full((pad,), n, jnp.int32)])

    edge_agg = _make_edge_agg(n, e_pad)
    mlp = _make_mlp(n, bm)

    h0 = x
    h1 = jnp.zeros((n, FH), jnp.float32)
    layer_params = [params["conv1"], params["conv2"], params["conv3"],
                    params["conv4"], params["conv4"], params["conv4"]]
    hs = []
    for li, p in enumerate(layer_params):
        z0, z1 = edge_agg(h0, h1, src, dst)
        w1, b1 = _fold_bn(p, f_in if li == 0 else 2 * FH)
        h0, h1 = mlp(z0, z1, w1, b1, p["W2"], p["b2"].reshape(1, -1))
        hs.extend([h0, h1])

    d = 2 * FH * 6
    pool_head = _make_pool_head(n, bm, g, d)
    batch_r = batch.reshape(n // bm, 1, bm)
    oh, ols = pool_head(batch_r, *hs,
                        params["lin1"]["W"], params["lin1"]["b"].reshape(1, -1),
                        params["lin2"]["W"], params["lin2"]["b"].reshape(1, -1))
    return (oh, ols)
